# baseline (speedup 1.0000x reference)
import functools

import jax
import jax.numpy as jnp
import numpy as np
from jax import lax
from jax.experimental import pallas as pl
from jax.experimental.pallas import tpu as pltpu
from jax.experimental.pallas import tpu_sc as plsc

B = 32
N = 16384 * 3
L = 16
C = N // L
RADIX = 256
HIST = RADIX * L
MIN32 = np.int32(-2147483648)


def _addr(r):
    q = ((r >> 10) * 21846) >> 16
    return (r << 4) - q * (16 * C - 1)


def _unkeys(k):
    bits = k ^ (jnp.bitwise_not(k >> 31) | MIN32)
    return plsc.bitcast(bits, jnp.float32)


def _for_count(count, unroll, body_fn):
    def outer(jo, _):
        base = jo * unroll
        for u in range(unroll):
            body_fn(base + u)
        return 0

    lax.fori_loop(0, count // unroll, outer, 0)


def _emd_body(xb, yb, out, bufa, bufb, hist, accb, spill):
    c = lax.axis_index("c")
    s = lax.axis_index("s")
    wid = s * 2 + c
    lane = lax.broadcasted_iota(jnp.int32, (L,), 0)
    ones = jnp.ones((L,), jnp.int32)
    zero = jnp.zeros((L,), jnp.int32)

    def hist_and_scan(load_fn, shift):
        def zbody(j):
            hist[pl.ds(j * L, L)] = zero

        _for_count(HIST // L, 8, zbody)

        def hbody(j):
            k = load_fn(j)
            d = (k >> shift) & 0xFF
            cell = (d << 4) | lane
            plsc.addupdate_scatter(hist, [cell], ones)

        _for_count(C, 8, hbody)

        def pbody(i, carry):
            v = hist[pl.ds(i * L, L)]
            ex = plsc.cumsum(v) - v
            hist[pl.ds(i * L, L)] = ex + carry
            return carry + jnp.sum(v)

        lax.fori_loop(0, HIST // L, pbody, jnp.int32(0))

    def radix_pass(load_fn, dst, shift):
        hist_and_scan(load_fn, shift)

        def sbody(j):
            k = load_fn(j)
            d = (k >> shift) & 0xFF
            cell = (d << 4) | lane
            pos = plsc.load_gather(hist, [cell])
            plsc.store_scatter(dst, [_addr(pos)], k)
            plsc.addupdate_scatter(hist, [cell], ones)

        _for_count(C, 8, sbody)

    def load_a(j):
        return bufa[pl.ds(j * L, L)]

    def load_b(j):
        return bufb[pl.ds(j * L, L)]

    def dma_row_in(src_hbm):
        for comp in range(3):
            pltpu.sync_copy(
                src_hbm.at[pl.ds((comp * B + wid) * (N // 3), N // 3)],
                bufa.at[pl.ds(comp * (N // 3), N // 3)],
            )

    dma_row_in(xb)
    radix_pass(load_a, bufb, 0)
    radix_pass(load_b, bufa, 8)
    radix_pass(load_a, bufb, 16)
    radix_pass(load_b, bufa, 24)
    pltpu.sync_copy(bufa, spill.at[pl.ds(wid * N, N)])

    dma_row_in(yb)
    radix_pass(load_a, bufb, 0)
    radix_pass(load_b, bufa, 8)
    radix_pass(load_a, bufb, 16)
    hist_and_scan(load_b, 24)
    pltpu.sync_copy(spill.at[pl.ds(wid * N, N)], bufa)

    def fbody(j, acc):
        k = load_b(j)
        d = (k >> 24) & 0xFF
        cell = (d << 4) | lane
        pos = plsc.load_gather(hist, [cell])
        xk = plsc.load_gather(bufa, [_addr(pos)])
        plsc.addupdate_scatter(hist, [cell], ones)
        dd = _unkeys(xk) - _unkeys(k)
        return acc + dd * dd

    def fouter(jo, acc):
        for u in range(8):
            acc = fbody(jo * 8 + u, acc)
        return acc

    acc = lax.fori_loop(0, C // 8, fouter, jnp.zeros((L,), jnp.float32))
    accb[...] = acc
    pltpu.sync_copy(accb, out.at[pl.ds(wid * L, L)])


_emd_sc = functools.partial(
    pl.kernel,
    out_type=jax.ShapeDtypeStruct((B * L,), jnp.float32),
    mesh=plsc.VectorSubcoreMesh(
        core_axis_name="c", subcore_axis_name="s", num_cores=2, num_subcores=16
    ),
    compiler_params=pltpu.CompilerParams(needs_layout_passes=False),
    scratch_types=[
        pltpu.VMEM((N,), jnp.int32),
        pltpu.VMEM((N,), jnp.int32),
        pltpu.VMEM((HIST,), jnp.int32),
        pltpu.VMEM((L,), jnp.float32),
        pltpu.HBM((B * N,), jnp.int32),
    ],
)(_emd_body)


def kernel(x, y):
    def to_keys(v):
        k = lax.bitcast_convert_type(v, jnp.int32)
        k = k.transpose(2, 0, 1).reshape(-1)
        return k ^ ((k >> 31) | MIN32)

    partials = _emd_sc(to_keys(x), to_keys(y))
    return jnp.sum(partials) / jnp.float32(B * N)


# speedup vs baseline: 1.0214x; 1.0214x over previous
import functools

import jax
import jax.numpy as jnp
import numpy as np
from jax import lax
from jax.experimental import pallas as pl
from jax.experimental.pallas import tpu as pltpu
from jax.experimental.pallas import tpu_sc as plsc

B = 32
N = 16384 * 3
L = 16
C = N // L
CP = C + 1
W = L * CP
RADIX = 256
HIST = RADIX * L
MIN32 = np.int32(-2147483648)


def _addr(r):
    return r + (((r >> 10) * 21846) >> 16)


def _unkeys(k):
    bits = k ^ (jnp.bitwise_not(k >> 31) | MIN32)
    return plsc.bitcast(bits, jnp.float32)


def _for_count(count, unroll, body_fn):
    def outer(jo, _):
        base = jo * unroll
        for u in range(unroll):
            body_fn(base + u)
        return 0

    lax.fori_loop(0, count // unroll, outer, 0)


def _emd_body(xb, yb, out, bufa, bufb, hist, accb, spill):
    c = lax.axis_index("c")
    s = lax.axis_index("s")
    wid = s * 2 + c
    lane = lax.broadcasted_iota(jnp.int32, (L,), 0)
    ones = jnp.ones((L,), jnp.int32)
    zero = jnp.zeros((L,), jnp.int32)
    lanecp = lane * CP

    def hist_and_scan(load_fn, shift):
        def zbody(j):
            hist[pl.ds(j * L, L)] = zero

        _for_count(HIST // L, 8, zbody)

        def hbody(j):
            k = load_fn(j)
            d = (k >> shift) & 0xFF
            cell = (d << 4) | lane
            plsc.addupdate_scatter(hist, [cell], ones)

        _for_count(C, 8, hbody)

        def pbody(i, carry):
            v = hist[pl.ds(i * L, L)]
            ex = plsc.cumsum(v) - v
            hist[pl.ds(i * L, L)] = ex + carry
            return carry + jnp.sum(v)

        lax.fori_loop(0, HIST // L, pbody, jnp.int32(0))

    def radix_pass(load_fn, dst, shift):
        hist_and_scan(load_fn, shift)

        def sbody(j):
            k = load_fn(j)
            d = (k >> shift) & 0xFF
            cell = (d << 4) | lane
            pos = plsc.load_gather(hist, [cell])
            plsc.store_scatter(dst, [_addr(pos)], k)
            plsc.addupdate_scatter(hist, [cell], ones)

        _for_count(C, 8, sbody)

    def load_contig(j):
        return bufa[pl.ds(j * L, L)]

    def load_pad_a(j):
        return plsc.load_gather(bufa, [lanecp + j])

    def load_pad_b(j):
        return plsc.load_gather(bufb, [lanecp + j])

    def dma_row_in(src_hbm):
        for comp in range(3):
            pltpu.sync_copy(
                src_hbm.at[pl.ds((comp * B + wid) * (N // 3), N // 3)],
                bufa.at[pl.ds(comp * (N // 3), N // 3)],
            )

    dma_row_in(xb)
    radix_pass(load_contig, bufb, 0)
    radix_pass(load_pad_b, bufa, 8)
    radix_pass(load_pad_a, bufb, 16)
    radix_pass(load_pad_b, bufa, 24)
    pltpu.sync_copy(bufa, spill.at[pl.ds(wid * W, W)])

    dma_row_in(yb)
    radix_pass(load_contig, bufb, 0)
    radix_pass(load_pad_b, bufa, 8)
    radix_pass(load_pad_a, bufb, 16)
    hist_and_scan(load_pad_b, 24)
    pltpu.sync_copy(spill.at[pl.ds(wid * W, W)], bufa)

    def fbody(j, acc):
        k = load_pad_b(j)
        d = (k >> 24) & 0xFF
        cell = (d << 4) | lane
        pos = plsc.load_gather(hist, [cell])
        xk = plsc.load_gather(bufa, [_addr(pos)])
        plsc.addupdate_scatter(hist, [cell], ones)
        dd = _unkeys(xk) - _unkeys(k)
        return acc + dd * dd

    def fouter(jo, acc):
        for u in range(8):
            acc = fbody(jo * 8 + u, acc)
        return acc

    acc = lax.fori_loop(0, C // 8, fouter, jnp.zeros((L,), jnp.float32))
    accb[...] = acc
    pltpu.sync_copy(accb, out.at[pl.ds(wid * L, L)])


_emd_sc = functools.partial(
    pl.kernel,
    out_type=jax.ShapeDtypeStruct((B * L,), jnp.float32),
    mesh=plsc.VectorSubcoreMesh(
        core_axis_name="c", subcore_axis_name="s", num_cores=2, num_subcores=16
    ),
    compiler_params=pltpu.CompilerParams(needs_layout_passes=False),
    scratch_types=[
        pltpu.VMEM((W,), jnp.int32),
        pltpu.VMEM((W,), jnp.int32),
        pltpu.VMEM((HIST,), jnp.int32),
        pltpu.VMEM((L,), jnp.float32),
        pltpu.HBM((B * W,), jnp.int32),
    ],
)(_emd_body)


def kernel(x, y):
    def to_keys(v):
        k = lax.bitcast_convert_type(v, jnp.int32)
        k = k.transpose(2, 0, 1).reshape(-1)
        return k ^ ((k >> 31) | MIN32)

    partials = _emd_sc(to_keys(x), to_keys(y))
    return jnp.sum(partials) / jnp.float32(B * N)
